# baseline (speedup 1.0000x reference)
import jax
import jax.numpy as jnp
from jax import lax
from jax.experimental import pallas as pl
from jax.experimental.pallas import tpu as pltpu
from jax.experimental.pallas import tpu_sc as plsc

B = 1024
F = 26
LH = 50
D = 64
NC, NS = 2, 16
NW = NC * NS
BPW = B // NW
KV = D // 16
CPR = F // 2
NCHUNK = BPW * CPR
NBUF = 4


def _rsqrt_vec(s_vec):
    i = plsc.bitcast(s_vec, jnp.int32)
    i = 0x5F3759DF - lax.shift_right_logical(i, 1)
    y = plsc.bitcast(i, jnp.float32)
    for _ in range(3):
        y = y * (1.5 - 0.5 * s_vec * y * y)
    return y


def _sc_body(uidx_hbm, sidx_hbm, utab_hbm, stab_hbm, out_hbm,
             uidx_v, sidx_v, sidx_sh, ulists, lists, ustage, accs, bufs,
             sem_u, sem_g, sem_o):
    cid = lax.axis_index("c")
    sid = lax.axis_index("s")
    wid = cid * NS + sid
    base = wid * BPW
    @pl.when(sid == 0)
    def _():
        pltpu.sync_copy(sidx_hbm.at[:, :, pl.ds(cid * (NS * BPW), NS * BPW)],
                        sidx_sh)

    pltpu.sync_copy(uidx_hbm.at[:, pl.ds(base, BPW)], uidx_v)
    plsc.subcore_barrier()
    pltpu.sync_copy(sidx_sh.at[:, :, pl.ds(sid * BPW, BPW)], sidx_v)

    iota = lax.iota(jnp.int32, 16)

    def build_seq_lists(b):
        slot = lax.rem(b, 2)
        b_v = jnp.full((16,), b, jnp.int32)

        slot_v = jnp.full((16,), slot, jnp.int32)

        def one_field(f, carry):
            f_v = jnp.full((16,), f, jnp.int32)
            for g4 in range(4):
                l_v = 16 * g4 + iota
                if g4 == 3:
                    mask = l_v < LH
                    vals = plsc.load_gather(sidx_v, [f_v, l_v, b_v], mask=mask)
                    plsc.store_scatter(lists, [slot_v, f_v, l_v], vals,
                                       mask=mask)
                else:
                    vals = plsc.load_gather(sidx_v, [f_v, l_v, b_v])
                    lists[slot, f, pl.ds(16 * g4, 16)] = vals
            return carry

        lax.fori_loop(0, F, one_field, 0)

    def build_ulist(b):
        slot = lax.rem(b, 2)
        b_v = jnp.full((16,), b, jnp.int32)
        for g2 in range(2):
            l_v = 16 * g2 + iota
            mask = l_v < F if g2 == 1 else None
            vals = plsc.load_gather(uidx_v, [l_v, b_v], mask=mask)
            ulists[slot, pl.ds(16 * g2, 16)] = vals

    def start_user(b):
        pltpu.async_copy(utab_hbm.at[ulists.at[lax.rem(b, 2), pl.ds(0, F)]],
                         ustage.at[lax.rem(b, 2)], sem_u.at[lax.rem(b, 2)])

    def start_chunk(b, c, slot):
        bslot = lax.rem(b, 2)
        pltpu.async_copy(stab_hbm.at[lists.at[bslot, 2 * c]],
                         bufs.at[slot, pl.ds(0, LH)], sem_g.at[slot])
        pltpu.async_copy(stab_hbm.at[lists.at[bslot, 2 * c + 1]],
                         bufs.at[slot, pl.ds(LH, LH)], sem_g.at[slot])

    for b in range(2):
        build_seq_lists(b)
        build_ulist(b)
        start_user(b)
    for j in range(NBUF):
        start_chunk(0, j, j)

    def chunk_step(g, sq_in):
        b = g // CPR
        c = g - b * CPR
        slot = lax.rem(g, NBUF)
        par = lax.rem(b, 2)

        @pl.when(c == 0)
        def _():
            @pl.when(b >= 2)
            def _():
                pltpu.make_async_copy(out_hbm.at[0], accs.at[0],
                                      sem_o.at[par]).wait()

            pltpu.make_async_copy(utab_hbm.at[pl.ds(0, F)], ustage.at[0],
                                  sem_u.at[par]).wait()

            @pl.when((b >= 1) & (b < BPW - 1))
            def _():
                build_seq_lists(b + 1)

            @pl.when(b < BPW - 2)
            def _():
                build_ulist(b + 2)

        pltpu.make_async_copy(stab_hbm.at[pl.ds(0, 2 * LH)], bufs.at[slot],
                              sem_g.at[slot]).wait()

        sq = jnp.where(c == 0, jnp.zeros((16,), jnp.float32), sq_in)
        for half in range(2):
            f = 2 * c + half
            v = [ustage[par, f, pl.ds(k * 16, 16)] for k in range(KV)]
            for l in range(LH):
                for k in range(KV):
                    v[k] = v[k] + bufs[slot, half * LH + l, pl.ds(k * 16, 16)]
            col = f * D
            for k in range(KV):
                accs[par, pl.ds(col + k * 16, 16)] = v[k]
                sq = sq + v[k] * v[k]

        @pl.when(g < NCHUNK - NBUF)
        def _():
            g2 = g + NBUF
            b2 = g2 // CPR
            c2 = g2 - b2 * CPR
            start_chunk(b2, c2, slot)

        @pl.when(c == CPR - 1)
        def _():
            @pl.when(b < BPW - 2)
            def _():
                start_user(b + 2)

            s = jnp.maximum(jnp.sum(sq), 1e-24)
            y = _rsqrt_vec(jnp.full((16,), s, jnp.float32))

            def scale(j, carry):
                accs[par, pl.ds(j * 16, 16)] = accs[par, pl.ds(j * 16, 16)] * y
                return carry

            lax.fori_loop(0, F * KV, scale, 0)
            pltpu.async_copy(accs.at[par], out_hbm.at[base + b], sem_o.at[par])

        return sq

    lax.fori_loop(0, NCHUNK, chunk_step, jnp.zeros((16,), jnp.float32))
    pltpu.make_async_copy(out_hbm.at[0], accs.at[0], sem_o.at[0]).wait()
    pltpu.make_async_copy(out_hbm.at[0], accs.at[0], sem_o.at[1]).wait()


@jax.jit
def kernel(user_idx, seq_idx, user_table, seq_table):
    mesh = plsc.VectorSubcoreMesh(core_axis_name="c", subcore_axis_name="s")
    run = pl.kernel(
        _sc_body,
        out_type=jax.ShapeDtypeStruct((B, F * D), jnp.float32),
        mesh=mesh,
        scratch_types=[
            pltpu.VMEM((F, BPW), jnp.int32),
            pltpu.VMEM((F, LH, BPW), jnp.int32),
            pltpu.VMEM_SHARED((F, LH, NS * BPW), jnp.int32),
            pltpu.VMEM((2, F), jnp.int32),
            pltpu.VMEM((2, F, LH), jnp.int32),
            pltpu.VMEM((2, F, D), jnp.float32),
            pltpu.VMEM((2, F * D), jnp.float32),
            pltpu.VMEM((NBUF, 2 * LH, D), jnp.float32),
            pltpu.SemaphoreType.DMA((2,)),
            pltpu.SemaphoreType.DMA((NBUF,)),
            pltpu.SemaphoreType.DMA((2,)),
        ],
        compiler_params=pltpu.CompilerParams(
            use_tc_tiling_on_sc=False, needs_layout_passes=False),
    )
    return run(user_idx.T, seq_idx.transpose(1, 2, 0), user_table, seq_table)


# speedup vs baseline: 1.0171x; 1.0171x over previous
import jax
import jax.numpy as jnp
from jax import lax
from jax.experimental import pallas as pl
from jax.experimental.pallas import tpu as pltpu
from jax.experimental.pallas import tpu_sc as plsc

B = 1024
F = 26
LH = 50
D = 64
NC, NS = 2, 16
NW = NC * NS
BPW = B // NW
KV = D // 16
CPR = F // 2
NCHUNK = BPW * CPR
NBUF = 4


def _rsqrt_vec(s_vec):
    i = plsc.bitcast(s_vec, jnp.int32)
    i = 0x5F3759DF - lax.shift_right_logical(i, 1)
    y = plsc.bitcast(i, jnp.float32)
    for _ in range(3):
        y = y * (1.5 - 0.5 * s_vec * y * y)
    return y


def _sc_body(uidx_hbm, sidx_hbm, utab_hbm, stab_hbm, out_hbm,
             uidx_v, sidx_v, sidx_sh, ulists, lists, ustage, accs, bufs,
             sem_u, sem_g, sem_o):
    cid = lax.axis_index("c")
    sid = lax.axis_index("s")
    wid = cid * NS + sid
    base = wid * BPW
    @pl.when(sid == 0)
    def _():
        pltpu.sync_copy(sidx_hbm.at[:, :, pl.ds(cid * (NS * BPW), NS * BPW)],
                        sidx_sh)

    pltpu.sync_copy(uidx_hbm.at[:, pl.ds(base, BPW)], uidx_v)
    plsc.subcore_barrier()
    pltpu.sync_copy(sidx_sh.at[:, :, pl.ds(sid * BPW, BPW)], sidx_v)

    iota = lax.iota(jnp.int32, 16)

    def build_seq_lists(b):
        slot = lax.rem(b, 2)
        b_v = jnp.full((16,), b, jnp.int32)

        slot_v = jnp.full((16,), slot, jnp.int32)

        def one_field(f, carry):
            f_v = jnp.full((16,), f, jnp.int32)
            for g4 in range(4):
                l_v = 16 * g4 + iota
                if g4 == 3:
                    mask = l_v < LH
                    vals = plsc.load_gather(sidx_v, [f_v, l_v, b_v], mask=mask)
                    plsc.store_scatter(lists, [slot_v, f_v, l_v], vals,
                                       mask=mask)
                else:
                    vals = plsc.load_gather(sidx_v, [f_v, l_v, b_v])
                    lists[slot, f, pl.ds(16 * g4, 16)] = vals
            return carry

        return one_field

    def build_seq_lists_all(b):
        lax.fori_loop(0, F, build_seq_lists(b), 0)

    def repack_pair(b, c):
        one_field = build_seq_lists(b)
        one_field(2 * c, 0)
        one_field(2 * c + 1, 0)

    def build_ulist(b):
        slot = lax.rem(b, 2)
        b_v = jnp.full((16,), b, jnp.int32)
        for g2 in range(2):
            l_v = 16 * g2 + iota
            mask = l_v < F if g2 == 1 else None
            vals = plsc.load_gather(uidx_v, [l_v, b_v], mask=mask)
            ulists[slot, pl.ds(16 * g2, 16)] = vals

    def start_user(b):
        pltpu.async_copy(utab_hbm.at[ulists.at[lax.rem(b, 2), pl.ds(0, F)]],
                         ustage.at[lax.rem(b, 2)], sem_u.at[lax.rem(b, 2)])

    def start_chunk(b, c, slot):
        bslot = lax.rem(b, 2)
        pltpu.async_copy(stab_hbm.at[lists.at[bslot, 2 * c]],
                         bufs.at[slot, pl.ds(0, LH)], sem_g.at[slot])
        pltpu.async_copy(stab_hbm.at[lists.at[bslot, 2 * c + 1]],
                         bufs.at[slot, pl.ds(LH, LH)], sem_g.at[slot])

    for b in range(2):
        build_seq_lists_all(b)
        build_ulist(b)
        start_user(b)
    for j in range(NBUF):
        start_chunk(0, j, j)

    def chunk_step(g, sq_in):
        b = g // CPR
        c = g - b * CPR
        slot = lax.rem(g, NBUF)
        par = lax.rem(b, 2)

        @pl.when(c == 0)
        def _():
            @pl.when(b >= 2)
            def _():
                pltpu.make_async_copy(out_hbm.at[0], accs.at[0],
                                      sem_o.at[par]).wait()

            pltpu.make_async_copy(utab_hbm.at[pl.ds(0, F)], ustage.at[0],
                                  sem_u.at[par]).wait()

            @pl.when(b < BPW - 2)
            def _():
                build_ulist(b + 2)

        pltpu.make_async_copy(stab_hbm.at[pl.ds(0, 2 * LH)], bufs.at[slot],
                              sem_g.at[slot]).wait()

        sq = jnp.where(c == 0, jnp.zeros((16,), jnp.float32), sq_in)
        for half in range(2):
            f = 2 * c + half
            v = [ustage[par, f, pl.ds(k * 16, 16)] for k in range(KV)]
            for l in range(LH):
                for k in range(KV):
                    v[k] = v[k] + bufs[slot, half * LH + l, pl.ds(k * 16, 16)]
            col = f * D
            for k in range(KV):
                accs[par, pl.ds(col + k * 16, 16)] = v[k]
                sq = sq + v[k] * v[k]

        @pl.when(g < NCHUNK - NBUF)
        def _():
            g2 = g + NBUF
            b2 = g2 // CPR
            c2 = g2 - b2 * CPR
            start_chunk(b2, c2, slot)

        @pl.when((b >= 1) & (b < BPW - 1))
        def _():
            repack_pair(b + 1, c)

        @pl.when(c == CPR - 1)
        def _():
            @pl.when(b < BPW - 2)
            def _():
                start_user(b + 2)

            s = jnp.maximum(jnp.sum(sq), 1e-24)
            y = _rsqrt_vec(jnp.full((16,), s, jnp.float32))

            def scale(j, carry):
                accs[par, pl.ds(j * 16, 16)] = accs[par, pl.ds(j * 16, 16)] * y
                return carry

            lax.fori_loop(0, F * KV, scale, 0)
            pltpu.async_copy(accs.at[par], out_hbm.at[base + b], sem_o.at[par])

        return sq

    lax.fori_loop(0, NCHUNK, chunk_step, jnp.zeros((16,), jnp.float32))
    pltpu.make_async_copy(out_hbm.at[0], accs.at[0], sem_o.at[0]).wait()
    pltpu.make_async_copy(out_hbm.at[0], accs.at[0], sem_o.at[1]).wait()


@jax.jit
def kernel(user_idx, seq_idx, user_table, seq_table):
    mesh = plsc.VectorSubcoreMesh(core_axis_name="c", subcore_axis_name="s")
    run = pl.kernel(
        _sc_body,
        out_type=jax.ShapeDtypeStruct((B, F * D), jnp.float32),
        mesh=mesh,
        scratch_types=[
            pltpu.VMEM((F, BPW), jnp.int32),
            pltpu.VMEM((F, LH, BPW), jnp.int32),
            pltpu.VMEM_SHARED((F, LH, NS * BPW), jnp.int32),
            pltpu.VMEM((2, F), jnp.int32),
            pltpu.VMEM((2, F, LH), jnp.int32),
            pltpu.VMEM((2, F, D), jnp.float32),
            pltpu.VMEM((2, F * D), jnp.float32),
            pltpu.VMEM((NBUF, 2 * LH, D), jnp.float32),
            pltpu.SemaphoreType.DMA((2,)),
            pltpu.SemaphoreType.DMA((NBUF,)),
            pltpu.SemaphoreType.DMA((2,)),
        ],
        compiler_params=pltpu.CompilerParams(
            use_tc_tiling_on_sc=False, needs_layout_passes=False),
    )
    return run(user_idx.T, seq_idx.transpose(1, 2, 0), user_table, seq_table)


# speedup vs baseline: 1.1509x; 1.1316x over previous
import jax
import jax.numpy as jnp
from jax import lax
from jax.experimental import pallas as pl
from jax.experimental.pallas import tpu as pltpu
from jax.experimental.pallas import tpu_sc as plsc

B = 1024
F = 26
LH = 50
D = 64
NC, NS = 2, 16
NW = NC * NS
BPW = B // NW
KV = D // 16
CPR = F // 2
NCHUNK = BPW * CPR
NBUF = 6


def _rsqrt_vec(s_vec):
    i = plsc.bitcast(s_vec, jnp.int32)
    i = 0x5F3759DF - lax.shift_right_logical(i, 1)
    y = plsc.bitcast(i, jnp.float32)
    for _ in range(3):
        y = y * (1.5 - 0.5 * s_vec * y * y)
    return y


def _sc_body(uidx_hbm, sidx_hbm, utab_hbm, stab_hbm, out_hbm,
             uidx_v, sidx_v, ustage, accs, bufs,
             sem_u, sem_g, sem_o):
    wid = lax.axis_index("s") * NC + lax.axis_index("c")
    base = wid * BPW
    pltpu.sync_copy(uidx_hbm.at[pl.ds(base, BPW)], uidx_v)
    pltpu.sync_copy(sidx_hbm.at[pl.ds(base, BPW)], sidx_v)

    for b in range(2):
        pltpu.async_copy(utab_hbm.at[uidx_v.at[b]], ustage.at[b], sem_u.at[b])
    def start_chunk(b, c, slot):
        pltpu.async_copy(stab_hbm.at[sidx_v.at[b, 2 * c]],
                         bufs.at[slot, pl.ds(0, LH)], sem_g.at[slot])
        pltpu.async_copy(stab_hbm.at[sidx_v.at[b, 2 * c + 1]],
                         bufs.at[slot, pl.ds(LH, LH)], sem_g.at[slot])

    for j in range(NBUF):
        start_chunk(0, j, j)

    def chunk_step(g, sq_in):
        b = g // CPR
        c = g - b * CPR
        slot = lax.rem(g, NBUF)
        par = lax.rem(b, 2)

        @pl.when((c == 0) & (b >= 2))
        def _():
            pltpu.make_async_copy(out_hbm.at[0], accs.at[0], sem_o.at[par]).wait()

        @pl.when(c == 0)
        def _():
            pltpu.make_async_copy(utab_hbm.at[pl.ds(0, F)], ustage.at[0],
                                  sem_u.at[par]).wait()

        pltpu.make_async_copy(stab_hbm.at[pl.ds(0, 2 * LH)], bufs.at[slot],
                              sem_g.at[slot]).wait()

        sq = jnp.where(c == 0, jnp.zeros((16,), jnp.float32), sq_in)
        for half in range(2):
            f = 2 * c + half
            v = [ustage[par, f, pl.ds(k * 16, 16)] for k in range(KV)]
            for l in range(LH):
                for k in range(KV):
                    v[k] = v[k] + bufs[slot, half * LH + l, pl.ds(k * 16, 16)]
            col = f * D
            for k in range(KV):
                accs[par, pl.ds(col + k * 16, 16)] = v[k]
                sq = sq + v[k] * v[k]

        @pl.when(g < NCHUNK - NBUF)
        def _():
            g2 = g + NBUF
            b2 = g2 // CPR
            c2 = g2 - b2 * CPR
            start_chunk(b2, c2, slot)

        @pl.when((c == CPR - 1) & (b < BPW - 2))
        def _():
            pltpu.async_copy(utab_hbm.at[uidx_v.at[b + 2]], ustage.at[par],
                             sem_u.at[par])

        @pl.when(c == CPR - 1)
        def _():
            s = jnp.maximum(jnp.sum(sq), 1e-24)
            y = _rsqrt_vec(jnp.full((16,), s, jnp.float32))

            def scale(j, carry):
                accs[par, pl.ds(j * 16, 16)] = accs[par, pl.ds(j * 16, 16)] * y
                return carry

            lax.fori_loop(0, F * KV, scale, 0)
            pltpu.async_copy(accs.at[par], out_hbm.at[base + b], sem_o.at[par])

        return sq

    lax.fori_loop(0, NCHUNK, chunk_step, jnp.zeros((16,), jnp.float32))
    pltpu.make_async_copy(out_hbm.at[0], accs.at[0], sem_o.at[0]).wait()
    pltpu.make_async_copy(out_hbm.at[0], accs.at[0], sem_o.at[1]).wait()


@jax.jit
def kernel(user_idx, seq_idx, user_table, seq_table):
    mesh = plsc.VectorSubcoreMesh(core_axis_name="c", subcore_axis_name="s")
    run = pl.kernel(
        _sc_body,
        out_type=jax.ShapeDtypeStruct((B, F * D), jnp.float32),
        mesh=mesh,
        scratch_types=[
            pltpu.VMEM((BPW, F), jnp.int32),
            pltpu.VMEM((BPW, F, LH), jnp.int32),
            pltpu.VMEM((2, F, D), jnp.float32),
            pltpu.VMEM((2, F * D), jnp.float32),
            pltpu.VMEM((NBUF, 2 * LH, D), jnp.float32),
            pltpu.SemaphoreType.DMA((2,)),
            pltpu.SemaphoreType.DMA((NBUF,)),
            pltpu.SemaphoreType.DMA((2,)),
        ],
        compiler_params=pltpu.CompilerParams(
            use_tc_tiling_on_sc=False, needs_layout_passes=False),
    )
    return run(user_idx, seq_idx, user_table, seq_table)


# speedup vs baseline: 1.1539x; 1.0026x over previous
import jax
import jax.numpy as jnp
from jax import lax
from jax.experimental import pallas as pl
from jax.experimental.pallas import tpu as pltpu
from jax.experimental.pallas import tpu_sc as plsc

B = 1024
F = 26
LH = 50
D = 64
NC, NS = 2, 16
NW = NC * NS
BPW = B // NW
KV = D // 16
CPR = F // 2
NCHUNK = BPW * CPR
NBUF = 4


def _rsqrt_vec(s_vec):
    i = plsc.bitcast(s_vec, jnp.int32)
    i = 0x5F3759DF - lax.shift_right_logical(i, 1)
    y = plsc.bitcast(i, jnp.float32)
    for _ in range(3):
        y = y * (1.5 - 0.5 * s_vec * y * y)
    return y


def _sc_body(uidx_hbm, sidx_hbm, utab_hbm, stab_hbm, out_hbm,
             uidx_v, sidx_v, ustage, accs, bufs,
             sem_u, sem_g, sem_o):
    wid = lax.axis_index("s") * NC + lax.axis_index("c")
    base = wid * BPW
    pltpu.sync_copy(uidx_hbm.at[pl.ds(base, BPW)], uidx_v)
    pltpu.sync_copy(sidx_hbm.at[pl.ds(base, BPW)], sidx_v)

    for b in range(2):
        pltpu.async_copy(utab_hbm.at[uidx_v.at[b]], ustage.at[b], sem_u.at[b])
    def start_chunk(b, c, slot):
        pltpu.async_copy(stab_hbm.at[sidx_v.at[b, 2 * c]],
                         bufs.at[slot, pl.ds(0, LH)], sem_g.at[slot])
        pltpu.async_copy(stab_hbm.at[sidx_v.at[b, 2 * c + 1]],
                         bufs.at[slot, pl.ds(LH, LH)], sem_g.at[slot])

    for j in range(NBUF):
        start_chunk(0, j, j)

    def chunk_step(g, sq_in):
        b = g // CPR
        c = g - b * CPR
        slot = lax.rem(g, NBUF)
        par = lax.rem(b, 2)

        @pl.when((c == 0) & (b >= 2))
        def _():
            pltpu.make_async_copy(out_hbm.at[0], accs.at[0], sem_o.at[par]).wait()

        @pl.when(c == 0)
        def _():
            pltpu.make_async_copy(utab_hbm.at[pl.ds(0, F)], ustage.at[0],
                                  sem_u.at[par]).wait()

        pltpu.make_async_copy(stab_hbm.at[pl.ds(0, 2 * LH)], bufs.at[slot],
                              sem_g.at[slot]).wait()

        sq = jnp.where(c == 0, jnp.zeros((16,), jnp.float32), sq_in)
        for half in range(2):
            f = 2 * c + half
            v = [ustage[par, f, pl.ds(k * 16, 16)] for k in range(KV)]
            for l in range(LH):
                for k in range(KV):
                    v[k] = v[k] + bufs[slot, half * LH + l, pl.ds(k * 16, 16)]
            col = f * D
            for k in range(KV):
                accs[par, pl.ds(col + k * 16, 16)] = v[k]
                sq = sq + v[k] * v[k]

        @pl.when(g < NCHUNK - NBUF)
        def _():
            g2 = g + NBUF
            b2 = g2 // CPR
            c2 = g2 - b2 * CPR
            start_chunk(b2, c2, slot)

        @pl.when((c == CPR - 1) & (b < BPW - 2))
        def _():
            pltpu.async_copy(utab_hbm.at[uidx_v.at[b + 2]], ustage.at[par],
                             sem_u.at[par])

        @pl.when(c == CPR - 1)
        def _():
            s = jnp.maximum(jnp.sum(sq), 1e-24)
            y = _rsqrt_vec(jnp.full((16,), s, jnp.float32))

            def scale(j, carry):
                accs[par, pl.ds(j * 16, 16)] = accs[par, pl.ds(j * 16, 16)] * y
                return carry

            lax.fori_loop(0, F * KV, scale, 0)
            pltpu.async_copy(accs.at[par], out_hbm.at[base + b], sem_o.at[par])

        return sq

    lax.fori_loop(0, NCHUNK, chunk_step, jnp.zeros((16,), jnp.float32))
    pltpu.make_async_copy(out_hbm.at[0], accs.at[0], sem_o.at[0]).wait()
    pltpu.make_async_copy(out_hbm.at[0], accs.at[0], sem_o.at[1]).wait()


@jax.jit
def kernel(user_idx, seq_idx, user_table, seq_table):
    mesh = plsc.VectorSubcoreMesh(core_axis_name="c", subcore_axis_name="s")
    run = pl.kernel(
        _sc_body,
        out_type=jax.ShapeDtypeStruct((B, F * D), jnp.float32),
        mesh=mesh,
        scratch_types=[
            pltpu.VMEM((BPW, F), jnp.int32),
            pltpu.VMEM((BPW, F, LH), jnp.int32),
            pltpu.VMEM((2, F, D), jnp.float32),
            pltpu.VMEM((2, F * D), jnp.float32),
            pltpu.VMEM((NBUF, 2 * LH, D), jnp.float32),
            pltpu.SemaphoreType.DMA((2,)),
            pltpu.SemaphoreType.DMA((NBUF,)),
            pltpu.SemaphoreType.DMA((2,)),
        ],
        compiler_params=pltpu.CompilerParams(
            use_tc_tiling_on_sc=False, needs_layout_passes=False),
    )
    return run(user_idx, seq_idx, user_table, seq_table)
